# baseline (speedup 1.0000x reference)
import functools

import jax
import jax.numpy as jnp
from jax import lax
from jax.experimental import pallas as pl
from jax.experimental.pallas import tpu as pltpu
from jax.experimental.pallas import tpu_sc as plsc

_B, _L, _D = 4096, 200, 64
_N = _B * _L
_NC, _NS = 2, 16
_NW = _NC * _NS
_CHUNK = 128
_PER_W = _N // _NW
_K = _PER_W // _CHUNK
_NBUF = 8


def _gather_sc(idx3, table):
  mesh = plsc.VectorSubcoreMesh(core_axis_name="c", subcore_axis_name="s")

  @functools.partial(
      pl.kernel,
      mesh=mesh,
      compiler_params=pltpu.CompilerParams(use_tc_tiling_on_sc=False),
      out_type=jax.ShapeDtypeStruct((_N, _D), jnp.float32),
      scratch_types=[
          pltpu.VMEM((_K, _CHUNK), jnp.int32),
          pltpu.VMEM((_NBUF, _CHUNK, _D), jnp.float32),
          pltpu.SemaphoreType.DMA((_NBUF,)),
          pltpu.SemaphoreType.DMA((_NBUF,)),
      ],
  )
  def body(idx_hbm, table_hbm, out_hbm, idx_v, rows_v, gsem, ssem):
    wid = lax.axis_index("s") * _NC + lax.axis_index("c")
    base = wid * _PER_W
    pltpu.sync_copy(idx_hbm.at[wid], idx_v)

    def gather(j, b):
      return pltpu.make_async_copy(
          table_hbm.at[idx_v.at[j]], rows_v.at[b], gsem.at[b])

    def store(j, b):
      return pltpu.make_async_copy(
          rows_v.at[b], out_hbm.at[pl.ds(base + j * _CHUNK, _CHUNK)],
          ssem.at[b])

    for b in range(_NBUF):
      gather(b, b).start()

    def group(i, carry):
      j0 = i * _NBUF
      for b in range(_NBUF):
        gather(j0 + b, b).wait()
        store(j0 + b, b).start()
      for b in range(_NBUF):
        store(j0 + b, b).wait()
        gather(j0 + b + _NBUF, b).start()
      return carry

    lax.fori_loop(0, _K // _NBUF - 1, group, 0)

    j0 = _K - _NBUF
    for b in range(_NBUF):
      gather(j0 + b, b).wait()
      store(j0 + b, b).start()
    for b in range(_NBUF):
      store(j0 + b, b).wait()

  return body(idx3, table)


def kernel(input_ids, token_types, word_embeddings):
  del token_types
  idx3 = input_ids.reshape(_NW, _K, _CHUNK)
  out = _gather_sc(idx3, word_embeddings)
  return out.reshape(_B, _L, _D)
